# baseline (speedup 1.0000x reference)
import numpy as np
import jax
import jax.numpy as jnp
from jax.experimental import pallas as pl

B = 4
HIDDEN = 256
NPROP = 1024
SAMPLE = 4000
MAXNN = 20
KQ = 10
R2 = np.float32(np.float64(0.2) ** 2)

BLKR = 200
QBLK = 256
GAP_TAU = np.float32(0.01)
EIGH_CAP = 4096


def _mlp_kernel(x_ref, w1_ref, b1_ref, g1_ref, be1_ref,
                w2_ref, b2_ref, g2_ref, be2_ref,
                wq_ref, bq_ref, wc_ref, bc_ref, ws_ref, bs_ref, base_ref,
                ctr_ref, sz_ref, qs_ref):
    x = x_ref[...]
    h = jnp.dot(w1_ref[...], x, preferred_element_type=jnp.float32) + b1_ref[...]
    m = jnp.mean(h, axis=1, keepdims=True)
    v = jnp.mean((h - m) ** 2, axis=1, keepdims=True)
    h = (h - m) / jnp.sqrt(v + 1e-5) * g1_ref[...] + be1_ref[...]
    h = jnp.maximum(h, 0.0)
    h2 = jnp.dot(w2_ref[...], h, preferred_element_type=jnp.float32) + b2_ref[...]
    m2 = jnp.mean(h2, axis=1, keepdims=True)
    v2 = jnp.mean((h2 - m2) ** 2, axis=1, keepdims=True)
    h2 = (h2 - m2) / jnp.sqrt(v2 + 1e-5) * g2_ref[...] + be2_ref[...]
    h2 = jnp.maximum(h2, 0.0)
    qs_ref[...] = jnp.dot(wq_ref[...], h2, preferred_element_type=jnp.float32) + bq_ref[...]
    ctr_ref[...] = (jnp.dot(wc_ref[...], h2, preferred_element_type=jnp.float32)
                    + bc_ref[...] + base_ref[...])
    sz_ref[...] = jnp.dot(ws_ref[...], h2, preferred_element_type=jnp.float32) + bs_ref[...]


def _count_le(d2, t):
    return jnp.sum((d2 <= t).astype(jnp.float32), axis=1, keepdims=True)


def _kth_thresh(d2, k, hi0, iters):
    lo0 = jnp.zeros_like(hi0)

    def body(_, lohi):
        lo, hi = lohi
        mid = (lo + hi) * 0.5
        ge = _count_le(d2, mid) >= k
        return jnp.where(ge, lo, mid), jnp.where(ge, mid, hi)

    _, hi = jax.lax.fori_loop(0, iters, body, (lo0, hi0))
    return hi


def _jacobi_rot(app, aqq, apq, arp, arq, cols):
    safe = apq != 0.0
    denom = jnp.where(safe, 2.0 * apq, 1.0)
    tau = (aqq - app) / denom
    sg = jnp.where(tau >= 0.0, 1.0, -1.0)
    t = jnp.where(safe, sg / (jnp.abs(tau) + jnp.sqrt(1.0 + tau * tau)), 0.0)
    c = 1.0 / jnp.sqrt(1.0 + t * t)
    s = t * c
    app2 = app - t * apq
    aqq2 = aqq + t * apq
    arp2 = c * arp - s * arq
    arq2 = s * arp + c * arq
    new_cols = []
    for vp, vq in cols:
        new_cols.append((c * vp - s * vq, s * vp + c * vq))
    return app2, aqq2, arp2, arq2, new_cols


def _jacobi3_smallest(a00, a01, a02, a11, a12, a22):
    one = jnp.ones_like(a00)
    zero = jnp.zeros_like(a00)
    v00, v01, v02 = one, zero, zero
    v10, v11, v12 = zero, one, zero
    v20, v21, v22 = zero, zero, one
    for _ in range(4):
        a00, a11, a02, a12, cols = _jacobi_rot(
            a00, a11, a01, a02, a12, [(v00, v01), (v10, v11), (v20, v21)])
        (v00, v01), (v10, v11), (v20, v21) = cols
        a01 = zero
        a00, a22, a01, a12, cols = _jacobi_rot(
            a00, a22, a02, a01, a12, [(v00, v02), (v10, v12), (v20, v22)])
        (v00, v02), (v10, v12), (v20, v22) = cols
        a02 = zero
        a11, a22, a01, a02, cols = _jacobi_rot(
            a11, a22, a12, a01, a02, [(v01, v02), (v11, v12), (v21, v22)])
        (v01, v02), (v11, v12), (v21, v22) = cols
        a12 = zero
    d0, d1, d2 = a00, a11, a22
    c0 = (d0 <= d1) & (d0 <= d2)
    c1 = jnp.logical_not(c0) & (d1 <= d2)
    vx = jnp.where(c0, v00, jnp.where(c1, v01, v02))
    vy = jnp.where(c0, v10, jnp.where(c1, v11, v12))
    vz = jnp.where(c0, v20, jnp.where(c1, v21, v22))
    lmin = jnp.where(c0, d0, jnp.where(c1, d1, d2))
    lmax = jnp.maximum(d0, jnp.maximum(d1, d2))
    lmid = (d0 + d1 + d2) - lmin - lmax
    gap = (lmid - lmin) / jnp.maximum(jnp.abs(lmax), jnp.float32(1e-30))
    return vx, vy, vz, gap


def _stats_kernel(pct_ref, pcr_ref, out_ref):
    px = pct_ref[0, 0:1, :]
    py = pct_ref[0, 1:2, :]
    pz = pct_ref[0, 2:3, :]
    qx = pcr_ref[0, :, 0:1]
    qy = pcr_ref[0, :, 1:2]
    qz = pcr_ref[0, :, 2:3]
    dx = qx - px
    dy = qy - py
    dz = qz - pz
    d2 = (dx * dx + dy * dy) + dz * dz
    hi0 = jnp.full((d2.shape[0], 1), R2, dtype=jnp.float32)
    thr = _kth_thresh(d2, float(MAXNN), hi0, 30)
    mask = d2 <= thr
    cnt = jnp.sum(mask.astype(jnp.float32), axis=1, keepdims=True)
    s1x = jnp.sum(jnp.where(mask, px, 0.0), axis=1, keepdims=True)
    s1y = jnp.sum(jnp.where(mask, py, 0.0), axis=1, keepdims=True)
    s1z = jnp.sum(jnp.where(mask, pz, 0.0), axis=1, keepdims=True)
    mux = s1x / cnt
    muy = s1y / cnt
    muz = s1z / cnt
    ax = px - mux
    ay = py - muy
    az = pz - muz
    axb = ax.astype(jnp.bfloat16).astype(jnp.float32)
    ayb = ay.astype(jnp.bfloat16).astype(jnp.float32)
    azb = az.astype(jnp.bfloat16).astype(jnp.float32)

    def cov_entry(a, b):
        p = a * b
        return jnp.sum(jnp.where(mask, p, 0.0), axis=1, keepdims=True) / cnt

    cxx = cov_entry(axb, axb)
    cxy = cov_entry(axb, ayb)
    cxz = cov_entry(axb, azb)
    cyy = cov_entry(ayb, ayb)
    cyz = cov_entry(ayb, azb)
    czz = cov_entry(azb, azb)
    vx, vy, vz, gap = _jacobi3_smallest(cxx, cxy, cxz, cyy, cyz, czz)
    is2 = (cnt > 1.5) & (cnt < 2.5)
    hard = is2 | ((cnt > 2.5) & (gap < GAP_TAU))
    flag = hard.astype(jnp.float32)
    out_ref[0] = jnp.concatenate(
        [cxx, cxy, cxz, cyy, cyz, czz, vx, vy, vz, flag], axis=1)


def _combine_kernel(pct_ref, nt_ref, qc_ref, out_ref):
    px = pct_ref[0, 0:1, :]
    py = pct_ref[0, 1:2, :]
    pz = pct_ref[0, 2:3, :]
    cx = jnp.sum(px) / float(SAMPLE)
    cy = jnp.sum(py) / float(SAMPLE)
    cz = jnp.sum(pz) / float(SAMPLE)
    nx0 = nt_ref[0, 0:1, :]
    ny0 = nt_ref[0, 1:2, :]
    nz0 = nt_ref[0, 2:3, :]
    dot = ((px - cx) * nx0 + (py - cy) * ny0) + (pz - cz) * nz0
    rev = dot < 0.0
    nx = jnp.where(rev, nx0, -nx0)
    ny = jnp.where(rev, ny0, -ny0)
    nz = jnp.where(rev, nz0, -nz0)
    qx = qc_ref[0, :, 0:1]
    qy = qc_ref[0, :, 1:2]
    qz = qc_ref[0, :, 2:3]
    dx = qx - px
    dy = qy - py
    dz = qz - pz
    d2 = (dx * dx + dy * dy) + dz * dz
    hi0 = jnp.max(d2, axis=1, keepdims=True)
    thr = _kth_thresh(d2, float(KQ), hi0, 44)
    mask = (d2 <= thr).astype(jnp.float32)
    nmat = jnp.concatenate([nx, ny, nz], axis=0)
    sel = jax.lax.dot_general(mask, nmat, (((1,), (1,)), ((), ())),
                              preferred_element_type=jnp.float32)
    sel = sel / float(KQ)
    sx = sel[:, 0:1]
    sy = sel[:, 1:2]
    nrm = jnp.sqrt(sx * sx + sy * sy)
    ox = sx / nrm
    oy = sy / nrm
    oz = jnp.zeros_like(sx) / nrm
    res = jnp.concatenate([ox, oy, oz], axis=1)
    res = jnp.where(jnp.isnan(res), jnp.float32(1e-6), res)
    out_ref[0] = res


def kernel(net, base_xyz, point_clouds, quad_center,
           W1, b1, g1, beta1, W2, b2, g2, beta2, Wq, bq, Wc, bc, Ws, bs):
    f32 = jnp.float32
    L = B * NPROP
    xT = jnp.transpose(net, (1, 0, 2)).reshape(HIDDEN, L)
    baseT = jnp.transpose(base_xyz, (2, 0, 1)).reshape(3, L)
    ctrT, szT, qsT = pl.pallas_call(
        _mlp_kernel,
        out_shape=[jax.ShapeDtypeStruct((3, L), f32),
                   jax.ShapeDtypeStruct((2, L), f32),
                   jax.ShapeDtypeStruct((2, L), f32)],
    )(xT, W1, b1.reshape(-1, 1), g1.reshape(-1, 1), beta1.reshape(-1, 1),
      W2, b2.reshape(-1, 1), g2.reshape(-1, 1), beta2.reshape(-1, 1),
      Wq, bq.reshape(-1, 1), Wc, bc.reshape(-1, 1), Ws, bs.reshape(-1, 1), baseT)
    center = ctrT.reshape(3, B, NPROP).transpose(1, 2, 0)
    size = szT.reshape(2, B, NPROP).transpose(1, 2, 0)
    quad_scores = qsT.reshape(2, B, NPROP).transpose(1, 2, 0)

    pc = point_clouds[:, :SAMPLE, :]
    pcT = jnp.transpose(pc, (0, 2, 1))
    stats = pl.pallas_call(
        _stats_kernel,
        grid=(B, SAMPLE // BLKR),
        in_specs=[pl.BlockSpec((1, 3, SAMPLE), lambda b, i: (b, 0, 0)),
                  pl.BlockSpec((1, BLKR, 3), lambda b, i: (b, i, 0))],
        out_specs=pl.BlockSpec((1, BLKR, 10), lambda b, i: (b, i, 0)),
        out_shape=jax.ShapeDtypeStruct((B, SAMPLE, 10), f32),
    )(pcT, pc)
    sflat = stats.reshape(B * SAMPLE, 10)
    cxx, cxy, cxz, cyy, cyz, czz = (sflat[:, k] for k in range(6))
    row0 = jnp.stack([cxx, cxy, cxz], axis=-1)
    row1 = jnp.stack([cxy, cyy, cyz], axis=-1)
    row2 = jnp.stack([cxz, cyz, czz], axis=-1)
    cov = jnp.stack([row0, row1, row2], axis=-2)
    idx = jnp.nonzero(sflat[:, 9] > 0.5, size=EIGH_CAP, fill_value=0)[0]
    _, hvecs = jnp.linalg.eigh(cov[idx])
    n0 = sflat[:, 6:9].at[idx].set(hvecs[..., 0])
    n0T = jnp.transpose(n0.reshape(B, SAMPLE, 3), (0, 2, 1))

    local_normals = pl.pallas_call(
        _combine_kernel,
        grid=(B, NPROP // QBLK),
        in_specs=[pl.BlockSpec((1, 3, SAMPLE), lambda b, i: (b, 0, 0)),
                  pl.BlockSpec((1, 3, SAMPLE), lambda b, i: (b, 0, 0)),
                  pl.BlockSpec((1, QBLK, 3), lambda b, i: (b, i, 0))],
        out_specs=pl.BlockSpec((1, QBLK, 3), lambda b, i: (b, i, 0)),
        out_shape=jax.ShapeDtypeStruct((B, NPROP, 3), f32),
    )(pcT, n0T, quad_center)

    return (center, size, quad_scores, local_normals)


# speedup vs baseline: 1.1379x; 1.1379x over previous
import numpy as np
import jax
import jax.numpy as jnp
from jax.experimental import pallas as pl

B = 4
HIDDEN = 256
NPROP = 1024
SAMPLE = 4000
MAXNN = 20
KQ = 10
R2 = np.float32(np.float64(0.2) ** 2)
R2_BITS = int(R2.view(np.int32))
MAXF_BITS = int(np.float32(np.finfo(np.float32).max).view(np.int32))

BLKR = 200
QBLK = 256
GAP_TAU = np.float32(0.01)
EIGH_CAP = 3584


def _mlp_kernel(x_ref, w1_ref, b1_ref, g1_ref, be1_ref,
                w2_ref, b2_ref, g2_ref, be2_ref,
                wq_ref, bq_ref, wc_ref, bc_ref, ws_ref, bs_ref, base_ref,
                ctr_ref, sz_ref, qs_ref):
    x = x_ref[...]
    h = jnp.dot(w1_ref[...], x, preferred_element_type=jnp.float32) + b1_ref[...]
    m = jnp.mean(h, axis=1, keepdims=True)
    v = jnp.mean((h - m) ** 2, axis=1, keepdims=True)
    h = (h - m) / jnp.sqrt(v + 1e-5) * g1_ref[...] + be1_ref[...]
    h = jnp.maximum(h, 0.0)
    h2 = jnp.dot(w2_ref[...], h, preferred_element_type=jnp.float32) + b2_ref[...]
    m2 = jnp.mean(h2, axis=1, keepdims=True)
    v2 = jnp.mean((h2 - m2) ** 2, axis=1, keepdims=True)
    h2 = (h2 - m2) / jnp.sqrt(v2 + 1e-5) * g2_ref[...] + be2_ref[...]
    h2 = jnp.maximum(h2, 0.0)
    qs_ref[...] = jnp.dot(wq_ref[...], h2, preferred_element_type=jnp.float32) + bq_ref[...]
    ctr_ref[...] = (jnp.dot(wc_ref[...], h2, preferred_element_type=jnp.float32)
                    + bc_ref[...] + base_ref[...])
    sz_ref[...] = jnp.dot(ws_ref[...], h2, preferred_element_type=jnp.float32) + bs_ref[...]


def _kth_thresh_bits(d2i, k, hi0, iters):
    hi0 = jnp.full((d2i.shape[0], 1), hi0, dtype=jnp.int32)
    lo0 = jnp.full_like(hi0, -1)

    def body(_, lohi):
        lo, hi = lohi
        mid = lo + (hi - lo) // 2
        cnt = jnp.sum((d2i <= mid).astype(jnp.float32), axis=1, keepdims=True)
        ge = cnt >= k
        return jnp.where(ge, lo, mid), jnp.where(ge, mid, hi)

    _, hi = jax.lax.fori_loop(0, iters, body, (lo0, hi0))
    return hi


def _jacobi_rot(app, aqq, apq, arp, arq, cols):
    safe = apq != 0.0
    denom = jnp.where(safe, 2.0 * apq, 1.0)
    tau = (aqq - app) / denom
    sg = jnp.where(tau >= 0.0, 1.0, -1.0)
    t = jnp.where(safe, sg / (jnp.abs(tau) + jnp.sqrt(1.0 + tau * tau)), 0.0)
    c = 1.0 / jnp.sqrt(1.0 + t * t)
    s = t * c
    app2 = app - t * apq
    aqq2 = aqq + t * apq
    arp2 = c * arp - s * arq
    arq2 = s * arp + c * arq
    new_cols = []
    for vp, vq in cols:
        new_cols.append((c * vp - s * vq, s * vp + c * vq))
    return app2, aqq2, arp2, arq2, new_cols


def _jacobi3_smallest(a00, a01, a02, a11, a12, a22):
    one = jnp.ones_like(a00)
    zero = jnp.zeros_like(a00)
    v00, v01, v02 = one, zero, zero
    v10, v11, v12 = zero, one, zero
    v20, v21, v22 = zero, zero, one
    for _ in range(4):
        a00, a11, a02, a12, cols = _jacobi_rot(
            a00, a11, a01, a02, a12, [(v00, v01), (v10, v11), (v20, v21)])
        (v00, v01), (v10, v11), (v20, v21) = cols
        a01 = zero
        a00, a22, a01, a12, cols = _jacobi_rot(
            a00, a22, a02, a01, a12, [(v00, v02), (v10, v12), (v20, v22)])
        (v00, v02), (v10, v12), (v20, v22) = cols
        a02 = zero
        a11, a22, a01, a02, cols = _jacobi_rot(
            a11, a22, a12, a01, a02, [(v01, v02), (v11, v12), (v21, v22)])
        (v01, v02), (v11, v12), (v21, v22) = cols
        a12 = zero
    d0, d1, d2 = a00, a11, a22
    c0 = (d0 <= d1) & (d0 <= d2)
    c1 = jnp.logical_not(c0) & (d1 <= d2)
    vx = jnp.where(c0, v00, jnp.where(c1, v01, v02))
    vy = jnp.where(c0, v10, jnp.where(c1, v11, v12))
    vz = jnp.where(c0, v20, jnp.where(c1, v21, v22))
    lmin = jnp.where(c0, d0, jnp.where(c1, d1, d2))
    lmax = jnp.maximum(d0, jnp.maximum(d1, d2))
    lmid = (d0 + d1 + d2) - lmin - lmax
    gap = (lmid - lmin) / jnp.maximum(jnp.abs(lmax), jnp.float32(1e-30))
    return vx, vy, vz, gap


def _stats_kernel(pct_ref, pcr_ref, out_ref):
    px = pct_ref[0, 0:1, :]
    py = pct_ref[0, 1:2, :]
    pz = pct_ref[0, 2:3, :]
    qx = pcr_ref[0, :, 0:1]
    qy = pcr_ref[0, :, 1:2]
    qz = pcr_ref[0, :, 2:3]
    dx = qx - px
    dy = qy - py
    dz = qz - pz
    d2 = (dx * dx + dy * dy) + dz * dz
    d2i = jax.lax.bitcast_convert_type(d2, jnp.int32)
    thr = _kth_thresh_bits(d2i, float(MAXNN), R2_BITS, 30)
    mask = d2i <= thr
    cnt = jnp.sum(mask.astype(jnp.float32), axis=1, keepdims=True)
    s1x = jnp.sum(jnp.where(mask, px, 0.0), axis=1, keepdims=True)
    s1y = jnp.sum(jnp.where(mask, py, 0.0), axis=1, keepdims=True)
    s1z = jnp.sum(jnp.where(mask, pz, 0.0), axis=1, keepdims=True)
    mux = s1x / cnt
    muy = s1y / cnt
    muz = s1z / cnt
    ax = px - mux
    ay = py - muy
    az = pz - muz
    axb = ax.astype(jnp.bfloat16).astype(jnp.float32)
    ayb = ay.astype(jnp.bfloat16).astype(jnp.float32)
    azb = az.astype(jnp.bfloat16).astype(jnp.float32)

    def cov_entry(a, b):
        p = a * b
        return jnp.sum(jnp.where(mask, p, 0.0), axis=1, keepdims=True) / cnt

    cxx = cov_entry(axb, axb)
    cxy = cov_entry(axb, ayb)
    cxz = cov_entry(axb, azb)
    cyy = cov_entry(ayb, ayb)
    cyz = cov_entry(ayb, azb)
    czz = cov_entry(azb, azb)
    vx, vy, vz, gap = _jacobi3_smallest(cxx, cxy, cxz, cyy, cyz, czz)
    is2 = (cnt > 1.5) & (cnt < 2.5)
    hard = is2 | ((cnt > 2.5) & (gap < GAP_TAU))
    flag = hard.astype(jnp.float32)
    out_ref[0] = jnp.concatenate(
        [cxx, cxy, cxz, cyy, cyz, czz, vx, vy, vz, flag], axis=1)


def _combine_kernel(pct_ref, nt_ref, qc_ref, out_ref):
    px = pct_ref[0, 0:1, :]
    py = pct_ref[0, 1:2, :]
    pz = pct_ref[0, 2:3, :]
    cx = jnp.sum(px) / float(SAMPLE)
    cy = jnp.sum(py) / float(SAMPLE)
    cz = jnp.sum(pz) / float(SAMPLE)
    nx0 = nt_ref[0, 0:1, :]
    ny0 = nt_ref[0, 1:2, :]
    nz0 = nt_ref[0, 2:3, :]
    dot = ((px - cx) * nx0 + (py - cy) * ny0) + (pz - cz) * nz0
    rev = dot < 0.0
    nx = jnp.where(rev, nx0, -nx0)
    ny = jnp.where(rev, ny0, -ny0)
    nz = jnp.where(rev, nz0, -nz0)
    qx = qc_ref[0, :, 0:1]
    qy = qc_ref[0, :, 1:2]
    qz = qc_ref[0, :, 2:3]
    dx = qx - px
    dy = qy - py
    dz = qz - pz
    d2 = (dx * dx + dy * dy) + dz * dz
    d2i = jax.lax.bitcast_convert_type(d2, jnp.int32)
    thr = _kth_thresh_bits(d2i, float(KQ), MAXF_BITS, 31)
    mask = (d2i <= thr).astype(jnp.float32)
    nmat = jnp.concatenate([nx, ny, nz], axis=0)
    sel = jax.lax.dot_general(mask, nmat, (((1,), (1,)), ((), ())),
                              preferred_element_type=jnp.float32)
    sel = sel / float(KQ)
    sx = sel[:, 0:1]
    sy = sel[:, 1:2]
    nrm = jnp.sqrt(sx * sx + sy * sy)
    ox = sx / nrm
    oy = sy / nrm
    oz = jnp.zeros_like(sx) / nrm
    res = jnp.concatenate([ox, oy, oz], axis=1)
    res = jnp.where(jnp.isnan(res), jnp.float32(1e-6), res)
    out_ref[0] = res


def kernel(net, base_xyz, point_clouds, quad_center,
           W1, b1, g1, beta1, W2, b2, g2, beta2, Wq, bq, Wc, bc, Ws, bs):
    f32 = jnp.float32
    L = B * NPROP
    xT = jnp.transpose(net, (1, 0, 2)).reshape(HIDDEN, L)
    baseT = jnp.transpose(base_xyz, (2, 0, 1)).reshape(3, L)
    ctrT, szT, qsT = pl.pallas_call(
        _mlp_kernel,
        out_shape=[jax.ShapeDtypeStruct((3, L), f32),
                   jax.ShapeDtypeStruct((2, L), f32),
                   jax.ShapeDtypeStruct((2, L), f32)],
    )(xT, W1, b1.reshape(-1, 1), g1.reshape(-1, 1), beta1.reshape(-1, 1),
      W2, b2.reshape(-1, 1), g2.reshape(-1, 1), beta2.reshape(-1, 1),
      Wq, bq.reshape(-1, 1), Wc, bc.reshape(-1, 1), Ws, bs.reshape(-1, 1), baseT)
    center = ctrT.reshape(3, B, NPROP).transpose(1, 2, 0)
    size = szT.reshape(2, B, NPROP).transpose(1, 2, 0)
    quad_scores = qsT.reshape(2, B, NPROP).transpose(1, 2, 0)

    pc = point_clouds[:, :SAMPLE, :]
    pcT = jnp.transpose(pc, (0, 2, 1))
    stats = pl.pallas_call(
        _stats_kernel,
        grid=(B, SAMPLE // BLKR),
        in_specs=[pl.BlockSpec((1, 3, SAMPLE), lambda b, i: (b, 0, 0)),
                  pl.BlockSpec((1, BLKR, 3), lambda b, i: (b, i, 0))],
        out_specs=pl.BlockSpec((1, BLKR, 10), lambda b, i: (b, i, 0)),
        out_shape=jax.ShapeDtypeStruct((B, SAMPLE, 10), f32),
    )(pcT, pc)
    sflat = stats.reshape(B * SAMPLE, 10)
    cxx, cxy, cxz, cyy, cyz, czz = (sflat[:, k] for k in range(6))
    row0 = jnp.stack([cxx, cxy, cxz], axis=-1)
    row1 = jnp.stack([cxy, cyy, cyz], axis=-1)
    row2 = jnp.stack([cxz, cyz, czz], axis=-1)
    cov = jnp.stack([row0, row1, row2], axis=-2)
    idx = jnp.nonzero(sflat[:, 9] > 0.5, size=EIGH_CAP, fill_value=0)[0]
    _, hvecs = jnp.linalg.eigh(cov[idx])
    n0 = sflat[:, 6:9].at[idx].set(hvecs[..., 0])
    n0T = jnp.transpose(n0.reshape(B, SAMPLE, 3), (0, 2, 1))

    local_normals = pl.pallas_call(
        _combine_kernel,
        grid=(B, NPROP // QBLK),
        in_specs=[pl.BlockSpec((1, 3, SAMPLE), lambda b, i: (b, 0, 0)),
                  pl.BlockSpec((1, 3, SAMPLE), lambda b, i: (b, 0, 0)),
                  pl.BlockSpec((1, QBLK, 3), lambda b, i: (b, i, 0))],
        out_specs=pl.BlockSpec((1, QBLK, 3), lambda b, i: (b, i, 0)),
        out_shape=jax.ShapeDtypeStruct((B, NPROP, 3), f32),
    )(pcT, n0T, quad_center)

    return (center, size, quad_scores, local_normals)
